# baseline (speedup 1.0000x reference)
import functools

import jax
import jax.numpy as jnp
from jax import lax
from jax.experimental import pallas as pl
from jax.experimental.pallas import tpu as pltpu

_KNN = 32
_SKIP = 0.8
_BISECT_ITERS = 18


def _graph_body(clen_ref, raw_ref, wenc_ref, benc_ref, wgl_ref, wg1_ref,
                bg1_ref, wg2_ref, bg2_ref, wout_ref, bout_ref, out_ref):
    b = pl.program_id(0)
    _, L, D = raw_ref.shape
    raw = raw_ref[0]
    clen = clen_ref[b]

    row_ids = lax.broadcasted_iota(jnp.int32, (L, 1), 0)
    col_ids = lax.broadcasted_iota(jnp.int32, (1, L), 1)
    rmask = (row_ids < clen).astype(jnp.float32)
    cmask = (col_ids < clen).astype(jnp.float32)

    ones_d = jnp.ones((D, 1), jnp.float32)
    nrm2 = lax.dot_general(raw * raw, ones_d, (((1,), (0,)), ((), ())),
                           preferred_element_type=jnp.float32)
    feat = raw * (1.0 / (jnp.sqrt(nrm2) + 1e-8))
    att = lax.dot_general(feat, feat, (((1,), (1,)), ((), ())),
                          preferred_element_type=jnp.float32)
    att = att * rmask * cmask

    lo0 = jnp.full((L, 1), -1.001, jnp.float32)
    hi0 = jnp.full((L, 1), 1.001, jnp.float32)
    ones_col = jnp.ones((L, 1), jnp.float32)

    def bisect_step(carry):
        lo, hi = carry
        mid = 0.5 * (lo + hi)
        cnt = jnp.sum(jnp.where(att >= mid, 1.0, 0.0), axis=1, keepdims=True)
        ge = cnt >= _KNN
        return jnp.where(ge, mid, lo), jnp.where(ge, hi, mid)

    carry = (lo0, hi0)
    for _ in range(_BISECT_ITERS):
        carry = bisect_step(carry)
    t_row = carry[0]

    kept = jnp.where(att >= t_row, att, 0.0)
    adj0 = 0.5 * (kept + kept.T)
    deg = jnp.sum(adj0, axis=1, keepdims=True)
    dinv = jnp.where(deg > 0.0, 1.0 / jnp.sqrt(deg + 1e-8), 0.0)
    init_adj = adj0 * dinv * dinv.T

    acc = jnp.zeros((L, L), jnp.float32)
    n_pers = wgl_ref.shape[0]
    for p in range(n_pers):
        wp = wgl_ref[p:p + 1, :]
        wf = raw * wp
        wn2 = lax.dot_general(wf * wf, ones_d, (((1,), (0,)), ((), ())),
                              preferred_element_type=jnp.float32)
        nwf = wf * (1.0 / (jnp.sqrt(wn2) + 1e-8))
        acc = acc + lax.dot_general(nwf, nwf, (((1,), (1,)), ((), ())),
                                    preferred_element_type=jnp.float32)
    att_l = acc * (1.0 / n_pers)
    att_l = jnp.where(att_l > 0.0, att_l, 0.0) * rmask * cmask
    learned = att_l / (jnp.sum(att_l, axis=1, keepdims=True) + 1e-8)

    adj = _SKIP * init_adj + (1.0 - _SKIP) * learned

    ctx = jnp.tanh(jnp.dot(raw, wenc_ref[...],
                           preferred_element_type=jnp.float32) + benc_ref[...])
    h = jnp.dot(adj, ctx, preferred_element_type=jnp.float32)
    h = jax.nn.relu(jnp.dot(h, wg1_ref[...],
                            preferred_element_type=jnp.float32) + bg1_ref[...])
    h = jnp.dot(adj, h, preferred_element_type=jnp.float32)
    h = jnp.dot(h, wg2_ref[...],
                preferred_element_type=jnp.float32) + bg2_ref[...]
    out = (jnp.dot(h, wout_ref[...],
                   preferred_element_type=jnp.float32) + bout_ref[...]) * rmask
    out_ref[0] = out


def kernel(context_vec, context_len, W_enc, b_enc, W_gl, W_g1, b_g1, W_g2,
           b_g2, W_out, b_out):
    B, L, D = context_vec.shape
    OUT = W_out.shape[1]
    rep = lambda b: (0, 0)
    small = lambda a: pl.BlockSpec(a.shape, rep)
    b_enc2, b_g12, b_g22, b_out2 = (x.reshape(1, -1)
                                    for x in (b_enc, b_g1, b_g2, b_out))
    return pl.pallas_call(
        _graph_body,
        grid=(B,),
        in_specs=[
            pl.BlockSpec(memory_space=pltpu.SMEM),
            pl.BlockSpec((1, L, D), lambda b: (b, 0, 0)),
            small(W_enc), small(b_enc2), small(W_gl),
            small(W_g1), small(b_g12), small(W_g2), small(b_g22),
            small(W_out), small(b_out2),
        ],
        out_specs=pl.BlockSpec((1, L, OUT), lambda b: (b, 0, 0)),
        out_shape=jax.ShapeDtypeStruct((B, L, OUT), jnp.float32),
    )(context_len, context_vec, W_enc, b_enc2, W_gl, W_g1, b_g12, W_g2,
      b_g22, W_out, b_out2)


# speedup vs baseline: 1.0862x; 1.0862x over previous
import jax
import jax.numpy as jnp
from jax import lax
from jax.experimental import pallas as pl
from jax.experimental.pallas import tpu as pltpu

_KNN = 32
_SKIP = 0.8
_BISECT_ITERS = 20


def _graph_body(clen_ref, raw_ref, wenc_ref, benc_ref, wgl_ref, wg1_ref,
                bg1_ref, wg2_ref, bg2_ref, wout_ref, bout_ref, out_ref):
    b = pl.program_id(0)
    _, L, D = raw_ref.shape
    raw = raw_ref[0]
    clen = clen_ref[b]

    row_ids = lax.broadcasted_iota(jnp.int32, (L, 1), 0)
    col_ids = lax.broadcasted_iota(jnp.int32, (1, L), 1)
    rmask = (row_ids < clen).astype(jnp.float32)
    cmask = (col_ids < clen).astype(jnp.float32)

    nrm = jnp.sqrt(jnp.sum(raw * raw, axis=1, keepdims=True))
    feat = raw / (nrm + 1e-8)
    att = lax.dot_general(feat, feat, (((1,), (1,)), ((), ())),
                          preferred_element_type=jnp.float32)
    att = att * rmask * cmask

    lo0 = jnp.full((L, 1), -1.001, jnp.float32)
    hi0 = jnp.full((L, 1), 1.001, jnp.float32)

    def bisect_step(carry):
        lo, hi = carry
        mid = 0.5 * (lo + hi)
        cnt = jnp.sum(jnp.where(att >= mid, 1.0, 0.0), axis=1, keepdims=True)
        ge = cnt >= _KNN
        return jnp.where(ge, mid, lo), jnp.where(ge, hi, mid)

    carry = (lo0, hi0)
    for _ in range(_BISECT_ITERS):
        carry = bisect_step(carry)
    t_row = carry[0]

    kept = jnp.where(att >= t_row, att, 0.0)
    adj0 = 0.5 * (kept + kept.T)
    deg = jnp.sum(adj0, axis=1, keepdims=True)
    dinv = jnp.where(deg > 0.0, 1.0 / jnp.sqrt(deg + 1e-8), 0.0)
    init_adj = adj0 * dinv * dinv.T

    acc = jnp.zeros((L, L), jnp.float32)
    n_pers = wgl_ref.shape[0]
    for p in range(n_pers):
        wp = wgl_ref[p:p + 1, :]
        wf = raw * wp
        wn = jnp.sqrt(jnp.sum(wf * wf, axis=1, keepdims=True))
        nwf = wf / (wn + 1e-8)
        acc = acc + lax.dot_general(nwf, nwf, (((1,), (1,)), ((), ())),
                                    preferred_element_type=jnp.float32)
    att_l = acc * (1.0 / n_pers)
    att_l = jnp.where(att_l > 0.0, att_l, 0.0) * rmask * cmask
    learned = att_l / (jnp.sum(att_l, axis=1, keepdims=True) + 1e-8)

    adj = _SKIP * init_adj + (1.0 - _SKIP) * learned

    ctx = jnp.tanh(jnp.dot(raw, wenc_ref[...],
                           preferred_element_type=jnp.float32) + benc_ref[...])
    h = jnp.dot(adj, ctx, preferred_element_type=jnp.float32)
    h = jax.nn.relu(jnp.dot(h, wg1_ref[...],
                            preferred_element_type=jnp.float32) + bg1_ref[...])
    h = jnp.dot(adj, h, preferred_element_type=jnp.float32)
    h = jnp.dot(h, wg2_ref[...],
                preferred_element_type=jnp.float32) + bg2_ref[...]
    out = (jnp.dot(h, wout_ref[...],
                   preferred_element_type=jnp.float32) + bout_ref[...]) * rmask
    out_ref[0] = out


def kernel(context_vec, context_len, W_enc, b_enc, W_gl, W_g1, b_g1, W_g2,
           b_g2, W_out, b_out):
    B, L, D = context_vec.shape
    OUT = W_out.shape[1]
    rep = lambda b: (0, 0)
    small = lambda a: pl.BlockSpec(a.shape, rep)
    b_enc2, b_g12, b_g22, b_out2 = (x.reshape(1, -1)
                                    for x in (b_enc, b_g1, b_g2, b_out))
    return pl.pallas_call(
        _graph_body,
        grid=(B,),
        in_specs=[
            pl.BlockSpec(memory_space=pltpu.SMEM),
            pl.BlockSpec((1, L, D), lambda b: (b, 0, 0)),
            small(W_enc), small(b_enc2), small(W_gl),
            small(W_g1), small(b_g12), small(W_g2), small(b_g22),
            small(W_out), small(b_out2),
        ],
        out_specs=pl.BlockSpec((1, L, OUT), lambda b: (b, 0, 0)),
        out_shape=jax.ShapeDtypeStruct((B, L, OUT), jnp.float32),
    )(context_len, context_vec, W_enc, b_enc2, W_gl, W_g1, b_g12, W_g2,
      b_g22, W_out, b_out2)
